# baseline (speedup 1.0000x reference)
import functools

import jax
import jax.numpy as jnp
from jax import lax
from jax.experimental import pallas as pl
from jax.experimental.pallas import tpu as pltpu
from jax.experimental.pallas import tpu_sc as plsc

BATCH = 16384
EMB = 32
NC = 2
NS = 16
NW = NC * NS
BPW = BATCH // NW
CHUNK = 128
NCHUNK = BPW // CHUNK
LANES = 16
ROWS_PER_STEP = 8

_mesh = plsc.VectorSubcoreMesh(core_axis_name="c", subcore_axis_name="s")


@functools.partial(
    pl.kernel,
    mesh=_mesh,
    compiler_params=pltpu.CompilerParams(use_tc_tiling_on_sc=False),
    out_type=jax.ShapeDtypeStruct((BATCH, EMB), jnp.float32),
    scratch_types=[
        pltpu.VMEM((NCHUNK, CHUNK), jnp.int32),
        pltpu.VMEM((NCHUNK, CHUNK), jnp.int32),
        pltpu.VMEM((BPW, EMB), jnp.float32),
        pltpu.VMEM((BPW, EMB), jnp.float32),
        pltpu.SemaphoreType.DMA,
        pltpu.SemaphoreType.DMA,
    ],
)
def _interac(first_hbm, second_hbm, w1_hbm, w2_hbm, out_hbm,
             idx1_v, idx2_v, rows1_v, rows2_v, sem1, sem2):
    wid = lax.axis_index("s") * NC + lax.axis_index("c")
    base = wid * BPW

    for c in range(NCHUNK):
        pltpu.sync_copy(first_hbm.at[pl.ds(base + c * CHUNK, CHUNK)],
                        idx1_v.at[c])
        pltpu.sync_copy(second_hbm.at[pl.ds(base + c * CHUNK, CHUNK)],
                        idx2_v.at[c])

    copies = []
    for c in range(NCHUNK):
        copies.append(pltpu.async_copy(
            w1_hbm.at[idx1_v.at[c]], rows1_v.at[pl.ds(c * CHUNK, CHUNK)],
            sem1))
        copies.append(pltpu.async_copy(
            w2_hbm.at[idx2_v.at[c]], rows2_v.at[pl.ds(c * CHUNK, CHUNK)],
            sem2))
    for cp in copies:
        cp.wait()

    def body(i, _):
        for r in range(ROWS_PER_STEP):
            row = i * ROWS_PER_STEP + r
            for j in range(EMB // LANES):
                sl = pl.ds(j * LANES, LANES)
                rows1_v[row, sl] = rows1_v[row, sl] * rows2_v[row, sl]
        return ()

    lax.fori_loop(0, BPW // ROWS_PER_STEP, body, ())

    pltpu.sync_copy(rows1_v, out_hbm.at[pl.ds(base, BPW)])


def kernel(first, second, W1, W2):
    return _interac(first, second, W1, W2)


# speedup vs baseline: 3.9461x; 3.9461x over previous
import functools

import jax
import jax.numpy as jnp
from jax import lax
from jax.experimental import pallas as pl
from jax.experimental.pallas import tpu as pltpu
from jax.experimental.pallas import tpu_sc as plsc

BATCH = 16384
EMB = 32
NC = 2
NS = 16
NW = NC * NS
BPW = BATCH // NW
LANES = 16
NBUF = 8
NGRP = BPW // LANES
TB_PER_W = BPW // 128

_mesh = plsc.VectorSubcoreMesh(core_axis_name="c", subcore_axis_name="s")

@functools.partial(
    pl.kernel,
    mesh=_mesh,
    compiler_params=pltpu.CompilerParams(needs_layout_passes=False),
    out_type=jax.ShapeDtypeStruct((4, BATCH // 128, 8, 128), jnp.float32),
    scratch_types=[
        pltpu.VMEM((BPW,), jnp.int32),
        pltpu.VMEM((BPW,), jnp.int32),
        pltpu.VMEM((NBUF, EMB, 128), jnp.float32),
        pltpu.VMEM((NBUF, EMB, 128), jnp.float32),
        pltpu.VMEM((4, TB_PER_W, 8, 128), jnp.float32),
        pltpu.SemaphoreType.DMA((2, NBUF)),
    ],
)
def _interac(first_hbm, second_hbm, w1t_hbm, w2t_hbm, out_hbm,
             idx1_v, idx2_v, g1_v, g2_v, ost_v, sems):
    wid = lax.axis_index("s") * NC + lax.axis_index("c")
    base = wid * BPW

    pltpu.sync_copy(first_hbm.at[pl.ds(base, BPW)], idx1_v)
    pltpu.sync_copy(second_hbm.at[pl.ds(base, BPW)], idx2_v)

    iota = lax.iota(jnp.int32, LANES)
    chi_lo = iota >> 3
    clo_lo = lax.bitwise_and(iota, jnp.int32(7))
    chi_hi = (iota + LANES) >> 3
    clo_hi = clo_lo

    def issue(gv1, gv2, j):
        s = j % NBUF
        w1 = pl.multiple_of(gv1[j], 128)
        w2 = pl.multiple_of(gv2[j], 128)
        pltpu.async_copy(w1t_hbm.at[:, pl.ds(w1, 128)], g1_v.at[s],
                         sems.at[0, s])
        pltpu.async_copy(w2t_hbm.at[:, pl.ds(w2, 128)], g2_v.at[s],
                         sems.at[1, s])

    def drain(s):
        pltpu.make_async_copy(w1t_hbm.at[:, pl.ds(0, 128)], g1_v.at[s],
                              sems.at[0, s]).wait()
        pltpu.make_async_copy(w2t_hbm.at[:, pl.ds(0, 128)], g2_v.at[s],
                              sems.at[1, s]).wait()

    def extract(b_prev, lv1, lv2, jlane, s):
        l1 = jnp.full((LANES,), lv1[jlane], jnp.int32)
        l2 = jnp.full((LANES,), lv2[jlane], jnp.int32)
        sv = jnp.full((LANES,), s, jnp.int32)
        v1a = plsc.load_gather(g1_v, [sv, iota, l1])
        v1b = plsc.load_gather(g1_v, [sv, iota + LANES, l1])
        v2a = plsc.load_gather(g2_v, [sv, iota, l2])
        v2b = plsc.load_gather(g2_v, [sv, iota + LANES, l2])
        tb = jnp.full((LANES,), b_prev >> 7, jnp.int32)
        blo = jnp.full((LANES,), b_prev & 127, jnp.int32)
        plsc.store_scatter(ost_v, [chi_lo, tb, clo_lo, blo], v1a * v2a)
        plsc.store_scatter(ost_v, [chi_hi, tb, clo_hi, blo], v1b * v2b)

    def body(g, carry):
        plv1, plv2 = carry
        i1 = idx1_v[pl.ds(g * LANES, LANES)]
        i2 = idx2_v[pl.ds(g * LANES, LANES)]
        gv1 = (i1 >> 7) * 128
        gv2 = (i2 >> 7) * 128
        lv1 = lax.bitwise_and(i1, jnp.int32(127))
        lv2 = lax.bitwise_and(i2, jnp.int32(127))
        for j in range(LANES):
            s = j % NBUF
            if j < NBUF:
                @pl.when(g > 0)
                def _():
                    drain(s)
                    extract(g * LANES + j - NBUF, plv1, plv2, j + NBUF, s)
            else:
                drain(s)
                extract(g * LANES + j - NBUF, lv1, lv2, j - NBUF, s)
            issue(gv1, gv2, j)
        return (lv1, lv2)

    zeros = jnp.zeros((LANES,), jnp.int32)
    lv1, lv2 = lax.fori_loop(0, NGRP, body, (zeros, zeros))

    for j in range(NBUF):
        drain(j)
        extract(BPW - NBUF + j, lv1, lv2, NBUF + j, j)

    for c_hi in range(4):
        pltpu.sync_copy(ost_v.at[c_hi],
                        out_hbm.at[c_hi, pl.ds(wid * TB_PER_W, TB_PER_W)])


def kernel(first, second, W1, W2):
    out4 = _interac(first, second, W1.T, W2.T)
    return out4.transpose(0, 2, 1, 3).reshape(EMB, BATCH).T
